# baseline (speedup 1.0000x reference)
import jax
import jax.numpy as jnp
from jax.experimental import pallas as pl

_B, _C, _H, _W = 16, 256, 32, 32
_HW = _H * _W
_N = _B * _HW
_COMMIT = 0.25
_SCALE = (1.0 + _COMMIT) / (_N * _C)
_BB = 4


def _loss_body(x_ref, w_ref, loss_ref):
    i = pl.program_id(0)
    d = x_ref[...] - w_ref[...]
    part = jnp.sum(d * d) * _SCALE

    @pl.when(i == 0)
    def _init():
        loss_ref[...] = jnp.zeros((1, 1), jnp.float32)

    loss_ref[...] += part.reshape(1, 1)


def _bcast_body(w_ref, out_ref, sidx_ref, cidx_ref):
    out_ref[0] = jnp.broadcast_to(w_ref[...].reshape(_C, 1), (_C, _HW))
    sidx_ref[...] = jnp.zeros((1, 1, _HW), jnp.int32)
    cidx_ref[...] = jnp.zeros((1, 1, _HW), jnp.int32)


def kernel(inputs, W_shape, W_color):
    x = inputs.reshape(_B, _C, _HW)
    w_cat = jnp.concatenate([W_shape[0], W_color[0]]).reshape(1, _C, 1)

    loss = pl.pallas_call(
        _loss_body,
        grid=(_B // _BB,),
        in_specs=[
            pl.BlockSpec((_BB, _C, _HW), lambda i: (i, 0, 0)),
            pl.BlockSpec((1, _C, 1), lambda i: (0, 0, 0)),
        ],
        out_specs=pl.BlockSpec((1, 1), lambda i: (0, 0)),
        out_shape=jax.ShapeDtypeStruct((1, 1), jnp.float32),
    )(x, w_cat)

    out, sidx, cidx = pl.pallas_call(
        _bcast_body,
        grid=(_B,),
        in_specs=[pl.BlockSpec((1, _C, 1), lambda i: (0, 0, 0))],
        out_specs=[
            pl.BlockSpec((1, _C, _HW), lambda i: (i, 0, 0)),
            pl.BlockSpec((1, 1, _HW), lambda i: (i, 0, 0)),
            pl.BlockSpec((1, 1, _HW), lambda i: (i, 0, 0)),
        ],
        out_shape=[
            jax.ShapeDtypeStruct((_B, _C, _HW), jnp.float32),
            jax.ShapeDtypeStruct((_B, 1, _HW), jnp.int32),
            jax.ShapeDtypeStruct((_B, 1, _HW), jnp.int32),
        ],
    )(w_cat)

    quantized = out.reshape(_B, _C, _H, _W)
    vq_loss = loss[0, 0]
    rep_loss = jnp.float32(0.0)
    return quantized, vq_loss, rep_loss, sidx.reshape(_N), cidx.reshape(_N)


# speedup vs baseline: 1.1075x; 1.1075x over previous
import jax
import jax.numpy as jnp
from jax.experimental import pallas as pl

_B, _C, _H, _W = 16, 256, 32, 32
_HW = _H * _W
_N = _B * _HW
_COMMIT = 0.25
_SCALE = (1.0 + _COMMIT) / (_N * _C)
_BB = 4


def _vq_body(x_ref, w_ref, out_ref, sidx_ref, cidx_ref, loss_ref):
    i = pl.program_id(0)
    x = x_ref[...]
    w = w_ref[...]
    d = x - w
    part = jnp.sum(d * d) * _SCALE

    @pl.when(i == 0)
    def _init():
        loss_ref[...] = jnp.zeros((1, 1), jnp.float32)

    loss_ref[...] += part.reshape(1, 1)
    out_ref[...] = jnp.broadcast_to(w, (_BB, _C, _HW))
    sidx_ref[...] = jnp.zeros((_BB, 1, _HW), jnp.int32)
    cidx_ref[...] = jnp.zeros((_BB, 1, _HW), jnp.int32)


def kernel(inputs, W_shape, W_color):
    x = inputs.reshape(_B, _C, _HW)
    w_cat = jnp.concatenate([W_shape[0], W_color[0]]).reshape(1, _C, 1)

    out, sidx, cidx, loss = pl.pallas_call(
        _vq_body,
        grid=(_B // _BB,),
        in_specs=[
            pl.BlockSpec((_BB, _C, _HW), lambda i: (i, 0, 0)),
            pl.BlockSpec((1, _C, 1), lambda i: (0, 0, 0)),
        ],
        out_specs=[
            pl.BlockSpec((_BB, _C, _HW), lambda i: (i, 0, 0)),
            pl.BlockSpec((_BB, 1, _HW), lambda i: (i, 0, 0)),
            pl.BlockSpec((_BB, 1, _HW), lambda i: (i, 0, 0)),
            pl.BlockSpec((1, 1), lambda i: (0, 0)),
        ],
        out_shape=[
            jax.ShapeDtypeStruct((_B, _C, _HW), jnp.float32),
            jax.ShapeDtypeStruct((_B, 1, _HW), jnp.int32),
            jax.ShapeDtypeStruct((_B, 1, _HW), jnp.int32),
            jax.ShapeDtypeStruct((1, 1), jnp.float32),
        ],
    )(x, w_cat)

    quantized = out.reshape(_B, _C, _H, _W)
    vq_loss = loss[0, 0]
    rep_loss = jnp.float32(0.0)
    return quantized, vq_loss, rep_loss, sidx.reshape(_N), cidx.reshape(_N)
